# baseline (speedup 1.0000x reference)
import functools

import jax
import jax.numpy as jnp
from jax import lax
from jax.experimental import pallas as pl
from jax.experimental.pallas import tpu as pltpu
from jax.experimental.pallas import tpu_sc as plsc

B = 16
TOTAL = 4096
SEQ_LEN = 128
CLS_ID = 101
SEP_ID = 102
EPW = 2
WIN = 136


def _extract_splat(vec, lane_splat):
    return vec.at[lane_splat].get(mode="promise_in_bounds")


def _body(flat1_hbm, cu1_hbm, flat2_hbm, cu2_hbm,
          ow_hbm, om_hbm, ot_hbm,
          f1_v, f2_v, cu1_v, cu2_v, bw_v, bm_v, bt_v,
          sem_in, sem_cu, sem_out):
    s = lax.axis_index("s")

    cp3 = pltpu.make_async_copy(cu1_hbm.at[pl.ds(0, 16)], cu1_v, sem_cu)
    cp4 = pltpu.make_async_copy(cu2_hbm.at[pl.ds(0, 16)], cu2_v, sem_cu)
    cp3.start()
    cp4.start()
    cp3.wait()
    cp4.wait()
    lanes = lax.iota(jnp.int32, 16)
    shift = jnp.minimum(lanes + 1, 15)
    c1lo = cu1_v[...]
    c2lo = cu2_v[...]
    c1hi = jnp.where(lanes == 15, TOTAL,
                     c1lo.at[shift].get(mode="promise_in_bounds"))
    c2hi = jnp.where(lanes == 15, TOTAL,
                     c2lo.at[shift].get(mode="promise_in_bounds"))

    len1 = c1hi - c1lo
    len2 = c2hi - c2lo
    avail = SEQ_LEN - 3
    cap1 = (avail + 1) // 2
    l1v = jnp.minimum(len1, jnp.maximum(cap1, avail - len2))
    l2v = jnp.minimum(len2, avail - l1v)

    cps = []
    bases = []
    for e in range(EPW):
        b = s * EPW + e
        lane_splat = jnp.full((16,), b, jnp.int32)
        s1 = _extract_splat(c1lo, lane_splat)
        s2 = _extract_splat(c2lo, lane_splat)
        base1 = pl.multiple_of(jnp.max(jnp.minimum(s1 & -8, TOTAL - WIN)), 8)
        base2 = pl.multiple_of(jnp.max(jnp.minimum(s2 & -8, TOTAL - WIN)), 8)
        cp1 = pltpu.make_async_copy(flat1_hbm.at[pl.ds(base1, WIN)],
                                    f1_v.at[pl.ds(e * WIN, WIN)], sem_in[e])
        cp2 = pltpu.make_async_copy(flat2_hbm.at[pl.ds(base2, WIN)],
                                    f2_v.at[pl.ds(e * WIN, WIN)], sem_in[e])
        cp1.start()
        cp2.start()
        cps.append((cp1, cp2))
        bases.append((s1, s2, base1, base2, lane_splat))

    for e in range(EPW):
        b = s * EPW + e
        s1, s2, base1, base2, lane_splat = bases[e]
        l1 = _extract_splat(l1v, lane_splat)
        l2 = _extract_splat(l2v, lane_splat)
        end = l1 + l2 + 2

        cps[e][0].wait()
        cps[e][1].wait()

        for j in range(SEQ_LEN // 16):
            idx = j * 16 + lanes
            g1 = jnp.clip(s1 + idx - 1 - base1, 0, WIN - 1)
            g2 = jnp.clip(s2 + (idx - l1 - 2) - base2, 0, WIN - 1)
            tok1 = plsc.load_gather(f1_v, [g1 + e * WIN])
            tok2 = plsc.load_gather(f2_v, [g2 + e * WIN])

            in1 = (idx >= 1) & (idx <= l1)
            in2 = (idx >= l1 + 2) & (idx <= end)
            is_sep = (idx == l1 + 1) | (idx == end)

            word = jnp.where(idx == 0, CLS_ID, 0)
            word = jnp.where(in1, tok1, word)
            word = jnp.where(in2, tok2, word)
            word = jnp.where(is_sep, SEP_ID, word)

            sl = pl.ds(e * SEQ_LEN + j * 16, 16)
            bw_v[sl] = word
            bm_v[sl] = (idx <= end).astype(jnp.int32)
            bt_v[sl] = in2.astype(jnp.int32)

        pltpu.make_async_copy(bw_v.at[pl.ds(e * SEQ_LEN, SEQ_LEN)], ow_hbm.at[b], sem_out).start()
        pltpu.make_async_copy(bm_v.at[pl.ds(e * SEQ_LEN, SEQ_LEN)], om_hbm.at[b], sem_out).start()
        pltpu.make_async_copy(bt_v.at[pl.ds(e * SEQ_LEN, SEQ_LEN)], ot_hbm.at[b], sem_out).start()

    for e in range(EPW):
        b = s * EPW + e
        pltpu.make_async_copy(bw_v.at[pl.ds(e * SEQ_LEN, SEQ_LEN)], ow_hbm.at[b], sem_out).wait()
        pltpu.make_async_copy(bm_v.at[pl.ds(e * SEQ_LEN, SEQ_LEN)], om_hbm.at[b], sem_out).wait()
        pltpu.make_async_copy(bt_v.at[pl.ds(e * SEQ_LEN, SEQ_LEN)], ot_hbm.at[b], sem_out).wait()


@jax.jit
def kernel(flat1, cu_seqlens1, flat2, cu_seqlens2):
    mesh = plsc.VectorSubcoreMesh(core_axis_name="c", subcore_axis_name="s",
                                  num_cores=1, num_subcores=B // EPW)
    run = functools.partial(
        pl.kernel,
        out_type=[jax.ShapeDtypeStruct((B, SEQ_LEN), jnp.int32)] * 3,
        mesh=mesh,
        compiler_params=pltpu.CompilerParams(needs_layout_passes=False),
        scratch_types=[
            pltpu.VMEM((EPW * WIN,), jnp.int32),
            pltpu.VMEM((EPW * WIN,), jnp.int32),
            pltpu.VMEM((B,), jnp.int32),
            pltpu.VMEM((B,), jnp.int32),
            pltpu.VMEM((EPW * SEQ_LEN,), jnp.int32),
            pltpu.VMEM((EPW * SEQ_LEN,), jnp.int32),
            pltpu.VMEM((EPW * SEQ_LEN,), jnp.int32),
            [pltpu.SemaphoreType.DMA] * EPW,
            pltpu.SemaphoreType.DMA,
            pltpu.SemaphoreType.DMA,
        ],
    )(_body)
    return tuple(run(flat1, cu_seqlens1, flat2, cu_seqlens2))


# speedup vs baseline: 1.0188x; 1.0188x over previous
import functools

import jax
import jax.numpy as jnp
from jax import lax
from jax.experimental import pallas as pl
from jax.experimental.pallas import tpu as pltpu
from jax.experimental.pallas import tpu_sc as plsc

B = 16
TOTAL = 4096
SEQ_LEN = 128
CLS_ID = 101
SEP_ID = 102
HALF = SEQ_LEN // 2


def _extract_splat(vec, lane_splat):
    return vec.at[lane_splat].get(mode="promise_in_bounds")


def _body(flat1_hbm, cu1_hbm, flat2_hbm, cu2_hbm,
          ow_hbm, om_hbm, ot_hbm,
          f1_v, f2_v, cu1_v, cu2_v, bw_v, bm_v, bt_v,
          sem_in, sem_cu, sem_out):
    s = lax.axis_index("s")

    cp3 = pltpu.make_async_copy(cu1_hbm.at[pl.ds(0, 16)], cu1_v, sem_cu)
    cp4 = pltpu.make_async_copy(cu2_hbm.at[pl.ds(0, 16)], cu2_v, sem_cu)
    cp3.start()
    cp4.start()
    cp3.wait()
    cp4.wait()
    lanes = lax.iota(jnp.int32, 16)
    shift = jnp.minimum(lanes + 1, 15)
    c1lo = cu1_v[...]
    c2lo = cu2_v[...]
    c1hi = jnp.where(lanes == 15, TOTAL,
                     c1lo.at[shift].get(mode="promise_in_bounds"))
    c2hi = jnp.where(lanes == 15, TOTAL,
                     c2lo.at[shift].get(mode="promise_in_bounds"))
    lane_splat = jnp.full((16,), s, jnp.int32)
    s1 = _extract_splat(c1lo, lane_splat)
    s2 = _extract_splat(c2lo, lane_splat)

    WIN = 136
    base1 = pl.multiple_of(jnp.max(jnp.minimum(s1 & -8, TOTAL - WIN)), 8)
    base2 = pl.multiple_of(jnp.max(jnp.minimum(s2 & -8, TOTAL - WIN)), 8)
    cp1 = pltpu.make_async_copy(flat1_hbm.at[pl.ds(base1, WIN)], f1_v, sem_in)
    cp2 = pltpu.make_async_copy(flat2_hbm.at[pl.ds(base2, WIN)], f2_v, sem_in)
    cp1.start()
    cp2.start()

    len1 = c1hi - c1lo
    len2 = c2hi - c2lo
    avail = SEQ_LEN - 3
    cap1 = (avail + 1) // 2
    l1v = jnp.minimum(len1, jnp.maximum(cap1, avail - len2))
    l2v = jnp.minimum(len2, avail - l1v)

    l1 = _extract_splat(l1v, lane_splat)
    l2 = _extract_splat(l2v, lane_splat)
    end = l1 + l2 + 2

    cp1.wait()
    cp2.wait()

    for j in range(SEQ_LEN // 16):
        idx = j * 16 + lanes
        g1 = jnp.clip(s1 + idx - 1 - base1, 0, WIN - 1)
        g2 = jnp.clip(s2 + (idx - l1 - 2) - base2, 0, WIN - 1)
        tok1 = plsc.load_gather(f1_v, [g1])
        tok2 = plsc.load_gather(f2_v, [g2])

        in1 = (idx >= 1) & (idx <= l1)
        in2 = (idx >= l1 + 2) & (idx <= end)
        is_sep = (idx == l1 + 1) | (idx == end)

        word = jnp.where(idx == 0, CLS_ID, 0)
        word = jnp.where(in1, tok1, word)
        word = jnp.where(in2, tok2, word)
        word = jnp.where(is_sep, SEP_ID, word)

        sl = pl.ds(j * 16, 16)
        bw_v[sl] = word
        bm_v[sl] = (idx <= end).astype(jnp.int32)
        bt_v[sl] = in2.astype(jnp.int32)

    ocp1 = pltpu.make_async_copy(bw_v, ow_hbm.at[s], sem_out)
    ocp2 = pltpu.make_async_copy(bm_v, om_hbm.at[s], sem_out)
    ocp3 = pltpu.make_async_copy(bt_v, ot_hbm.at[s], sem_out)
    ocp1.start()
    ocp2.start()
    ocp3.start()
    ocp1.wait()
    ocp2.wait()
    ocp3.wait()


@jax.jit
def kernel(flat1, cu_seqlens1, flat2, cu_seqlens2):
    mesh = plsc.VectorSubcoreMesh(core_axis_name="c", subcore_axis_name="s",
                                  num_cores=1)
    run = functools.partial(
        pl.kernel,
        out_type=[jax.ShapeDtypeStruct((B, SEQ_LEN), jnp.int32)] * 3,
        mesh=mesh,
        compiler_params=pltpu.CompilerParams(needs_layout_passes=False),
        scratch_types=[
            pltpu.VMEM((136,), jnp.int32),
            pltpu.VMEM((136,), jnp.int32),
            pltpu.VMEM((B,), jnp.int32),
            pltpu.VMEM((B,), jnp.int32),
            pltpu.VMEM((SEQ_LEN,), jnp.int32),
            pltpu.VMEM((SEQ_LEN,), jnp.int32),
            pltpu.VMEM((SEQ_LEN,), jnp.int32),
            pltpu.SemaphoreType.DMA,
            pltpu.SemaphoreType.DMA,
            pltpu.SemaphoreType.DMA,
        ],
    )(_body)
    return tuple(run(flat1, cu_seqlens1, flat2, cu_seqlens2))
